# Initial kernel scaffold; baseline (speedup 1.0000x reference)
import jax
import jax.numpy as jnp
from jax.experimental import pallas as pl


def kernel(target_surface_forms, target_priors, input_embeddings, bias):
    raise NotImplementedError("write your pallas kernel here")


# speedup vs baseline: 1.5669x; 1.5669x over previous
import functools

import jax
import jax.numpy as jnp
from jax import lax
from jax.experimental import pallas as pl
from jax.experimental.pallas import tpu as pltpu
from jax.experimental.pallas import tpu_sc as plsc

B, L = 16384, 16
V, D = 100000, 768

NC, NS = 2, 16
NW = NC * NS
B_PER_W = B // NW
CHUNK = 64
NCHUNK = B_PER_W // CHUNK
BIAS_W = 128
BIAS_ROWS = (V + BIAS_W - 1) // BIAS_W

_mesh = plsc.VectorSubcoreMesh(core_axis_name="c", subcore_axis_name="s")


@functools.partial(
    pl.kernel,
    mesh=_mesh,
    compiler_params=pltpu.CompilerParams(needs_layout_passes=False),
    out_type=(
        jax.ShapeDtypeStruct((B, D), jnp.float32),
        jax.ShapeDtypeStruct((B,), jnp.float32),
    ),
    scratch_types=[
        pltpu.VMEM((NCHUNK, CHUNK), jnp.int32),
        pltpu.VMEM((NCHUNK, CHUNK), jnp.int32),
        pltpu.VMEM((2, CHUNK, D), jnp.float32),
        pltpu.VMEM((2, CHUNK, BIAS_W), jnp.float32),
        pltpu.VMEM((B_PER_W,), jnp.float32),
        pltpu.SemaphoreType.DMA,
        pltpu.SemaphoreType.DMA,
        pltpu.SemaphoreType.DMA,
        pltpu.SemaphoreType.DMA,
    ],
)
def _gather_kernel(ids_hbm, rid_hbm, table_hbm, bias_hbm, out_hbm,
                   bias_out_hbm, idx_v, rid_v, rows_v, brows_v, bias_v,
                   sem0, sem1, semb0, semb1):
    wid = lax.axis_index("s") * NC + lax.axis_index("c")
    base = wid * B_PER_W

    pltpu.sync_copy(ids_hbm.at[wid], idx_v)
    pltpu.sync_copy(rid_hbm.at[wid], rid_v)

    sems = (sem0, sem1)
    bsems = (semb0, semb1)

    def start_gather(j):
        return pltpu.async_copy(
            table_hbm.at[idx_v.at[j]], rows_v.at[j % 2], sems[j % 2])

    def start_bias(j):
        return pltpu.async_copy(
            bias_hbm.at[rid_v.at[j]], brows_v.at[j % 2], bsems[j % 2])

    cur = start_gather(0)
    curb = start_bias(0)
    for j in range(NCHUNK):
        nxt = start_gather(j + 1) if j + 1 < NCHUNK else None
        nxtb = start_bias(j + 1) if j + 1 < NCHUNK else None
        cur.wait()
        pltpu.sync_copy(rows_v.at[j % 2],
                        out_hbm.at[pl.ds(base + j * CHUNK, CHUNK)])
        curb.wait()
        for g in range(CHUNK // 16):
            ids16 = idx_v[j, pl.ds(g * 16, 16)]
            offs = ids16 & (BIAS_W - 1)
            rows = lax.iota(jnp.int32, 16) + (g * 16)
            vals = plsc.load_gather(brows_v.at[j % 2], [rows, offs])
            bias_v[pl.ds(j * CHUNK + g * 16, 16)] = vals
        cur, curb = nxt, nxtb

    pltpu.sync_copy(bias_v, bias_out_hbm.at[pl.ds(base, B_PER_W)])


def kernel(target_surface_forms, target_priors, input_embeddings, bias):
    del target_priors
    ids = target_surface_forms[:, 0].astype(jnp.int32)
    rid = (ids >> 7).reshape(NW, NCHUNK, CHUNK)
    ids = ids.reshape(NW, NCHUNK, CHUNK)
    bias2d = jnp.pad(bias[:, 0], (0, BIAS_ROWS * BIAS_W - V))
    bias2d = bias2d.reshape(BIAS_ROWS, BIAS_W)
    emb, b = _gather_kernel(ids, rid, input_embeddings, bias2d)
    return emb, b
